# baseline (speedup 1.0000x reference)
import functools

import jax
import jax.numpy as jnp
from jax.experimental import pallas as pl

B = 2048
K = 128
D = 510
P = 332

BLK = 512


def _first_argmin_onehot(scores, k):
    iota = jax.lax.broadcasted_iota(jnp.int32, scores.shape, 1)
    m = jnp.min(scores, axis=1, keepdims=True)
    idx = jnp.min(jnp.where(scores == m, iota, k), axis=1, keepdims=True)
    return (iota == idx).astype(jnp.float32)


def _kern(z_ref, zp_ref, pos_ref, q_ref, qt_ref, pn_ref, out_ref):
    q = q_ref[...]
    qt = qt_ref[...]
    qn = jnp.sum(qt * qt, axis=0)[None, :]
    zb = z_ref[...]
    zpb = zp_ref[...]

    s1 = qn - 2.0 * jax.lax.dot_general(
        zb, qt, (((1,), (0,)), ((), ())),
        preferred_element_type=jnp.float32, precision=jax.lax.Precision.HIGHEST)
    s2 = qn - 2.0 * jax.lax.dot_general(
        zpb, qt, (((1,), (0,)), ((), ())),
        preferred_element_type=jnp.float32, precision=jax.lax.Precision.HIGHEST)

    oh = _first_argmin_onehot(s1, K) + _first_argmin_onehot(s2, K)
    zq_sum = jax.lax.dot_general(
        oh, q, (((1,), (0,)), ((), ())),
        preferred_element_type=jnp.float32)

    pos = pos_ref[...]
    piota = jax.lax.broadcasted_iota(jnp.int32, (BLK, P), 1)
    poh = (piota == pos).astype(jnp.float32)
    pe = jax.lax.dot_general(
        poh, pn_ref[...], (((1,), (0,)), ((), ())),
        preferred_element_type=jnp.float32)

    out_ref[...] = zq_sum + pe


@jax.jit
def kernel(z, z_pre, position_number, quantisation, phrase_number):
    pos2d = position_number.astype(jnp.int32).reshape(B, 1)
    qt = quantisation.T
    grid = B // BLK
    return pl.pallas_call(
        _kern,
        grid=(grid,),
        in_specs=[
            pl.BlockSpec((BLK, D), lambda i: (i, 0)),
            pl.BlockSpec((BLK, D), lambda i: (i, 0)),
            pl.BlockSpec((BLK, 1), lambda i: (i, 0)),
            pl.BlockSpec((K, D), lambda i: (0, 0)),
            pl.BlockSpec((D, K), lambda i: (0, 0)),
            pl.BlockSpec((P, D), lambda i: (0, 0)),
        ],
        out_specs=pl.BlockSpec((BLK, D), lambda i: (i, 0)),
        out_shape=jax.ShapeDtypeStruct((B, D), jnp.float32),
    )(z, z_pre, pos2d, quantisation, qt, phrase_number)
